# baseline (speedup 1.0000x reference)
import functools

import jax
import jax.numpy as jnp
from jax import lax
from jax.experimental import pallas as pl
from jax.experimental.pallas import tpu as pltpu
from jax.experimental.pallas import tpu_sc as plsc

N = 10000
NP = 10240
E = 640000
M = 8000
D_IN = 1000
H1 = 500
LAT = 64
DEC = 64
HID = 64
NCLS = 8

NCORES = 2
NTILES = 16
CHUNK = 128
ECHUNKS = E // CHUNK
ECHUNKS_PC = ECHUNKS // NCORES
NPT = NP // NTILES
NCHUNKS_N = NP // CHUNK


def _mm1_body(x_ref, w_ref, b_ref, u_ref, s_ref, ss_ref):
    i = pl.program_id(0)
    u = jnp.dot(x_ref[...], w_ref[...], preferred_element_type=jnp.float32,
                precision=lax.Precision.DEFAULT)
    u = u + b_ref[...]
    u_ref[...] = u

    @pl.when(i == 0)
    def _():
        s_ref[...] = jnp.zeros_like(s_ref)
        ss_ref[...] = jnp.zeros_like(ss_ref)

    s_ref[...] += jnp.sum(u, axis=0, keepdims=True)
    ss_ref[...] += jnp.sum(u * u, axis=0, keepdims=True)


def _mm1(x, w, b):
    bm = 1000
    grid = M // bm
    return pl.pallas_call(
        _mm1_body,
        grid=(grid,),
        in_specs=[
            pl.BlockSpec((bm, D_IN), lambda i: (i, 0)),
            pl.BlockSpec((D_IN, H1), lambda i: (0, 0)),
            pl.BlockSpec((1, H1), lambda i: (0, 0)),
        ],
        out_specs=[
            pl.BlockSpec((bm, H1), lambda i: (i, 0)),
            pl.BlockSpec((1, H1), lambda i: (0, 0)),
            pl.BlockSpec((1, H1), lambda i: (0, 0)),
        ],
        out_shape=[
            jax.ShapeDtypeStruct((M, H1), jnp.float32),
            jax.ShapeDtypeStruct((1, H1), jnp.float32),
            jax.ShapeDtypeStruct((1, H1), jnp.float32),
        ],
    )(x, w, b)


def _bn_scale(s, ss, g, be):
    mu = s * (1.0 / M)
    var = ss * (1.0 / M) - mu * mu
    sc = g * lax.rsqrt(var + 1e-5)
    t = be - mu * sc
    return sc, t


def _mm2_body(u_ref, w_ref, b_ref, g_ref, be_ref, s1_ref, ss1_ref,
              e_ref, s2_ref, ss2_ref):
    i = pl.program_id(0)
    sc, t = _bn_scale(s1_ref[...], ss1_ref[...], g_ref[...], be_ref[...])
    w = w_ref[...]
    bias = jnp.dot(t, w, preferred_element_type=jnp.float32) + b_ref[...]
    e = jnp.dot(u_ref[...] * sc, w, preferred_element_type=jnp.float32,
                precision=lax.Precision.DEFAULT) + bias
    e_ref[...] = e

    @pl.when(i == 0)
    def _():
        s2_ref[...] = jnp.zeros_like(s2_ref)
        ss2_ref[...] = jnp.zeros_like(ss2_ref)

    s2_ref[...] += jnp.sum(e, axis=0, keepdims=True)
    ss2_ref[...] += jnp.sum(e * e, axis=0, keepdims=True)


def _mm2(u, w, b, g, be, s1, ss1):
    bm = 1000
    grid = M // bm
    return pl.pallas_call(
        _mm2_body,
        grid=(grid,),
        in_specs=[
            pl.BlockSpec((bm, H1), lambda i: (i, 0)),
            pl.BlockSpec((H1, LAT), lambda i: (0, 0)),
            pl.BlockSpec((1, LAT), lambda i: (0, 0)),
            pl.BlockSpec((1, H1), lambda i: (0, 0)),
            pl.BlockSpec((1, H1), lambda i: (0, 0)),
            pl.BlockSpec((1, H1), lambda i: (0, 0)),
            pl.BlockSpec((1, H1), lambda i: (0, 0)),
        ],
        out_specs=[
            pl.BlockSpec((bm, LAT), lambda i: (i, 0)),
            pl.BlockSpec((1, LAT), lambda i: (0, 0)),
            pl.BlockSpec((1, LAT), lambda i: (0, 0)),
        ],
        out_shape=[
            jax.ShapeDtypeStruct((M, LAT), jnp.float32),
            jax.ShapeDtypeStruct((1, LAT), jnp.float32),
            jax.ShapeDtypeStruct((1, LAT), jnp.float32),
        ],
    )(u, w, b, g, be, s1, ss1)


_SGN = -2147483648
_LOW31 = 0x7FFFFFFF


def _order_stat(ksh, r):
    negm = jnp.where(ksh < 0, 1, 0).astype(jnp.int32)
    cneg128 = jnp.sum(negm, axis=0, keepdims=True)
    cneg = cneg128[:, :DEC] + cneg128[:, DEC:]
    in_negi = jnp.where(r < cneg, 1, 0).astype(jnp.int32)
    in_neg2i = jnp.concatenate([in_negi, in_negi], axis=1)
    grp = (in_neg2i * negm + (1 - in_neg2i) * (1 - negm)).astype(jnp.float32)
    radj = jnp.where(in_negi == 1, r, r - cneg).astype(jnp.float32)
    q = ksh & _LOW31
    p = jnp.zeros((1, DEC), jnp.int32)
    ones_row = jnp.ones((1, M // 2), jnp.float32)

    def step(i, carry):
        p, radj, cand = carry
        b = 30 - i
        bit = jnp.left_shift(jnp.int32(1), b)
        iszero = jnp.where((q & bit) == 0, 1.0, 0.0)
        m = cand * iszero
        c128 = jnp.dot(ones_row, m, preferred_element_type=jnp.float32)
        c0 = c128[:, :DEC] + c128[:, DEC:]
        take1 = radj >= c0
        take1f = jnp.where(take1, 1.0, 0.0)
        p = p | jnp.left_shift(jnp.where(take1, 1, 0).astype(jnp.int32), b)
        radj = radj - take1f * c0
        t2f = jnp.concatenate([take1f, take1f], axis=1)
        cand = cand * (iszero + t2f - 2.0 * iszero * t2f)
        return p, radj, cand

    p, radj, cand = lax.fori_loop(0, 31, step, (p, radj, grp))
    return jnp.where(in_negi == 1, p | _SGN, p)


def _mm3_body(e_ref, w_ref, b_ref, g_ref, be_ref, s2_ref, ss2_ref,
              dec_ref, med_ref):
    sc, t = _bn_scale(s2_ref[...], ss2_ref[...], g_ref[...], be_ref[...])
    w = w_ref[...]
    bias = jnp.dot(t, w, preferred_element_type=jnp.float32) + b_ref[...]
    dec = jnp.dot(e_ref[...] * sc, w, preferred_element_type=jnp.float32) + bias
    dec_ref[...] = dec

    bi = lax.bitcast_convert_type(dec, jnp.int32)
    ks = jnp.where(bi < 0, bi ^ _LOW31, bi)
    ksh = jnp.concatenate([ks[: M // 2], ks[M // 2:]], axis=1)
    k1 = _order_stat(ksh, M // 2 - 1)
    k1_2 = jnp.concatenate([k1, k1], axis=1)
    le128 = jnp.sum(jnp.where(ksh <= k1_2, 1, 0), axis=0, keepdims=True)
    cle = le128[:, :DEC] + le128[:, DEC:]
    big = jnp.where(ksh > k1_2, ksh, jnp.int32(2147483647))
    mn128 = jnp.min(big, axis=0, keepdims=True)
    mn = jnp.minimum(mn128[:, :DEC], mn128[:, DEC:])
    k2 = jnp.where(cle > M // 2, k1, mn)

    def unkey(k):
        vi = jnp.where(k < 0, k ^ _LOW31, k)
        return lax.bitcast_convert_type(vi, jnp.float32)

    med_ref[...] = (unkey(k1) + unkey(k2)) * 0.5


def _mm3(e, w, b, g, be, s2, ss2):
    return pl.pallas_call(
        _mm3_body,
        out_shape=[
            jax.ShapeDtypeStruct((M, DEC), jnp.float32),
            jax.ShapeDtypeStruct((1, DEC), jnp.float32),
        ],
    )(e, w, b, g, be, s2, ss2)


def _scale_body(do_ref, di_ref, g0_ref, g1_ref, fs_ref, no_ref, ni_ref):
    do = do_ref[...]
    di = di_ref[...]
    no = jnp.where(do > 0, lax.rsqrt(jnp.maximum(do, 1.0)), 0.0)
    ni = jnp.where(di > 0, lax.rsqrt(jnp.maximum(di, 1.0)), 0.0)
    no_ref[...] = no
    ni_ref[...] = ni
    fs_ref[...] = (g0_ref[...] + g1_ref[...]) * 0.5 * no


def _scale(do, di, g0, g1):
    return pl.pallas_call(
        _scale_body,
        out_shape=[
            jax.ShapeDtypeStruct((NP, DEC), jnp.float32),
            jax.ShapeDtypeStruct((NP, 1), jnp.float32),
            jax.ShapeDtypeStruct((NP, 1), jnp.float32),
        ],
    )(do, di, g0, g1)


def _l1_body(p0_ref, p1_ref, ni_ref, no_ref, w1_ref, b1_ref, w2_ref, y_ref):
    agg = (p0_ref[...] + p1_ref[...]) * ni_ref[...]
    h = jnp.dot(agg, w1_ref[...], preferred_element_type=jnp.float32)
    h = jax.nn.relu(h + b1_ref[...])
    y_ref[...] = jnp.dot(h * no_ref[...], w2_ref[...],
                         preferred_element_type=jnp.float32)


def _l1(p0, p1, ni, no, w1, b1, w2p):
    return pl.pallas_call(
        _l1_body,
        out_shape=jax.ShapeDtypeStruct((NP, 16), jnp.float32),
    )(p0, p1, ni, no, w1, b1, w2p)


def _l2_body(q0_ref, q1_ref, ni_ref, b_ref, o_ref):
    o_ref[...] = (q0_ref[...] + q1_ref[...]) * ni_ref[...] + b_ref[...]


def _l2(q0, q1, ni, b):
    return pl.pallas_call(
        _l2_body,
        out_shape=jax.ShapeDtypeStruct((NP, 16), jnp.float32),
    )(q0, q1, ni, b)


_MESH = dict(core_axis_name="c", subcore_axis_name="s")
_SC_PARAMS = pltpu.CompilerParams(use_tc_tiling_on_sc=False)


def _deg_sc(ei3, ones128, zdeg):
    @functools.partial(
        pl.kernel,
        mesh=plsc.VectorSubcoreMesh(**_MESH),
        compiler_params=_SC_PARAMS,
        out_type=jax.ShapeDtypeStruct((NCORES, NP), jnp.float32),
        scratch_types=[
            pltpu.VMEM((8, CHUNK), jnp.int32),
            pltpu.VMEM((1, CHUNK), jnp.int32),
            pltpu.VMEM((CHUNK,), jnp.float32),
            pltpu.VMEM((NPT,), jnp.float32),
            pltpu.VMEM_SHARED((NP,), jnp.float32),
            pltpu.SemaphoreType.DMA,
        ],
    )
    def k(ei_hbm, ones_hbm, z_hbm, out_hbm, idx_v, idx1_v, ones_v, buf_v,
          acc_sh, sem):
        c = lax.axis_index("c")
        s = lax.axis_index("s")
        pltpu.sync_copy(z_hbm.at[pl.ds(s * NPT, NPT)], buf_v)
        pltpu.sync_copy(buf_v, acc_sh.at[pl.ds(s * NPT, NPT)])
        pltpu.sync_copy(ones_hbm, ones_v)
        plsc.subcore_barrier()
        cbase = s * 312 + jnp.minimum(s, 8)

        def body(g, carry):
            gb = cbase + g * 8
            pltpu.sync_copy(ei_hbm.at[c, pl.ds(gb, 8)], idx_v)
            cps = [
                pltpu.async_copy(ones_v, acc_sh.at[idx_v.at[j]], sem, add=True)
                for j in range(8)
            ]
            for cp in cps:
                cp.wait()
            return carry

        lax.fori_loop(0, 39, body, 0)

        @pl.when(s < 8)
        def _():
            ch = cbase + 312
            pltpu.sync_copy(ei_hbm.at[c, pl.ds(ch, 1)], idx1_v)
            pltpu.sync_copy(ones_v, acc_sh.at[idx1_v.at[0]], add=True)

        plsc.subcore_barrier()
        pltpu.sync_copy(acc_sh.at[pl.ds(s * NPT, NPT)], buf_v)
        pltpu.sync_copy(buf_v, out_hbm.at[c, pl.ds(s * NPT, NPT)])

    return k(ei3, ones128, zdeg)


def _gather_sc(t0, t1, r03, r13):
    @functools.partial(
        pl.kernel,
        mesh=plsc.VectorSubcoreMesh(**_MESH),
        compiler_params=_SC_PARAMS,
        out_type=(
            jax.ShapeDtypeStruct((NP, DEC), jnp.float32),
            jax.ShapeDtypeStruct((NP, DEC), jnp.float32),
        ),
        scratch_types=[
            pltpu.VMEM((CHUNK,), jnp.int32),
            pltpu.VMEM((CHUNK,), jnp.int32),
            pltpu.VMEM((CHUNK, DEC), jnp.float32),
            pltpu.VMEM((CHUNK, DEC), jnp.float32),
            pltpu.SemaphoreType.DMA,
        ],
    )
    def k(t0_hbm, t1_hbm, r0_hbm, r1_hbm, g0_hbm, g1_hbm,
          i0_v, i1_v, b0_v, b1_v, sem):
        c = lax.axis_index("c")
        s = lax.axis_index("s")
        wid = s * NCORES + c
        nch = jnp.where(wid < 16, 3, 2)
        cbase = wid * 2 + jnp.minimum(wid, 16)

        def body(kk, carry):
            ch = cbase + kk
            pltpu.sync_copy(r0_hbm.at[ch], i0_v)
            pltpu.sync_copy(r1_hbm.at[ch], i1_v)
            pltpu.async_copy(t0_hbm.at[i0_v], b0_v, sem).wait()
            pltpu.async_copy(t1_hbm.at[i1_v], b1_v, sem).wait()
            pltpu.sync_copy(b0_v, g0_hbm.at[pl.ds(ch * CHUNK, CHUNK)])
            pltpu.sync_copy(b1_v, g1_hbm.at[pl.ds(ch * CHUNK, CHUNK)])
            return carry

        lax.fori_loop(0, nch, body, 0)

    return k(t0, t1, r03, r13)


def _edge_sc(x, ei3, zeros, d):
    @functools.partial(
        pl.kernel,
        mesh=plsc.VectorSubcoreMesh(**_MESH),
        compiler_params=_SC_PARAMS,
        out_type=jax.ShapeDtypeStruct((NCORES, NP, d), jnp.float32),
        scratch_types=[
            pltpu.VMEM((6, CHUNK), jnp.int32),
            pltpu.VMEM((6, CHUNK), jnp.int32),
            pltpu.VMEM((6, CHUNK, d), jnp.float32),
            pltpu.VMEM((NPT // 4, d), jnp.float32),
            pltpu.VMEM_SHARED((NP, d), jnp.float32),
            pltpu.SemaphoreType.DMA,
            pltpu.SemaphoreType.DMA,
        ],
    )
    def k(x_hbm, ei_hbm, z_hbm, out_hbm,
          src_v, dst_v, rows_v, buf_v, acc_sh, gsem, ssem):
        c = lax.axis_index("c")
        s = lax.axis_index("s")
        zr = NPT // 4
        for t in range(4):
            pltpu.sync_copy(z_hbm.at[pl.ds(s * NPT + t * zr, zr)], buf_v)
            pltpu.sync_copy(buf_v, acc_sh.at[pl.ds(s * NPT + t * zr, zr)])
        plsc.subcore_barrier()
        cbase = c * ECHUNKS_PC + s * 156 + jnp.minimum(s, 4)

        def body(g, carry):
            gb = cbase + g * 6
            pltpu.sync_copy(ei_hbm.at[0, pl.ds(gb, 6)], src_v)
            pltpu.sync_copy(ei_hbm.at[1, pl.ds(gb, 6)], dst_v)
            gcps = [
                pltpu.async_copy(x_hbm.at[src_v.at[j]], rows_v.at[j], gsem)
                for j in range(6)
            ]
            for cp in gcps:
                cp.wait()
            scps = [
                pltpu.async_copy(rows_v.at[j], acc_sh.at[dst_v.at[j]], ssem,
                                 add=True)
                for j in range(6)
            ]
            for cp in scps:
                cp.wait()
            return carry

        lax.fori_loop(0, 26, body, 0)

        @pl.when(s < 4)
        def _():
            ch = cbase + 156
            pltpu.sync_copy(ei_hbm.at[0, pl.ds(ch, 1)], src_v.at[pl.ds(0, 1)])
            pltpu.sync_copy(ei_hbm.at[1, pl.ds(ch, 1)], dst_v.at[pl.ds(0, 1)])
            pltpu.async_copy(x_hbm.at[src_v.at[0]], rows_v.at[0], gsem).wait()
            pltpu.async_copy(rows_v.at[0], acc_sh.at[dst_v.at[0]], ssem,
                             add=True).wait()

        plsc.subcore_barrier()
        for t in range(4):
            pltpu.sync_copy(acc_sh.at[pl.ds(s * NPT + t * zr, zr)], buf_v)
            pltpu.sync_copy(buf_v, out_hbm.at[c, pl.ds(s * NPT + t * zr, zr)])

    return k(x, ei3, zeros)


def kernel(x0, x1,
           W1_0, b1_0, g1_0, be1_0, W2_0, b2_0, g2_0, be2_0, Wd_0, bd_0,
           W1_1, b1_1, g1_1, be1_1, W2_1, b2_1, g2_1, be2_1, Wd_1, bd_1,
           Wg1, bg1, Wg2, bg2, edge_index, reindex0, reindex1):
    r2 = lambda v: v.reshape(1, -1)

    tables = []
    mods = (
        (x0, (W1_0, b1_0, g1_0, be1_0, W2_0, b2_0, g2_0, be2_0, Wd_0, bd_0)),
        (x1, (W1_1, b1_1, g1_1, be1_1, W2_1, b2_1, g2_1, be2_1, Wd_1, bd_1)),
    )
    for x, (W1, b1, g1, be1, W2, b2, g2, be2, Wd, bd) in mods:
        u, s1, ss1 = _mm1(x, W1, r2(b1))
        e, s2, ss2 = _mm2(u, W2, r2(b2), r2(g1), r2(be1), s1, ss1)
        dec, med = _mm3(e, Wd, r2(bd), r2(g2), r2(be2), s2, ss2)
        t = jnp.concatenate(
            [dec, jnp.broadcast_to(med, (N - M, DEC)),
             jnp.zeros((NP - N, DEC), jnp.float32)], axis=0)
        tables.append(t)

    ei3 = edge_index.reshape(2, ECHUNKS, CHUNK)
    pad_idx = jnp.zeros((NP - N,), jnp.int32)
    r03 = jnp.concatenate([reindex0, pad_idx]).reshape(NCHUNKS_N, CHUNK)
    r13 = jnp.concatenate([reindex1, pad_idx]).reshape(NCHUNKS_N, CHUNK)

    ones128 = jnp.ones((CHUNK,), jnp.float32)
    zdeg = jnp.zeros((NP,), jnp.float32)
    z64 = jnp.zeros((NP, DEC), jnp.float32)
    z16 = jnp.zeros((NP, 16), jnp.float32)

    deg = _deg_sc(ei3, ones128, zdeg)
    g0, g1 = _gather_sc(tables[0], tables[1], r03, r13)
    fs, no, ni = _scale(deg[0].reshape(NP, 1), deg[1].reshape(NP, 1), g0, g1)

    p = _edge_sc(fs, ei3, z64, DEC)

    Wg2p = jnp.concatenate(
        [Wg2, jnp.zeros((HID, 16 - NCLS), jnp.float32)], axis=1)
    bg2p = jnp.concatenate(
        [bg2, jnp.zeros((16 - NCLS,), jnp.float32)]).reshape(1, 16)

    y = _l1(p[0], p[1], ni, no, Wg1, r2(bg1), Wg2p)
    q = _edge_sc(y, ei3, z16, 16)
    out = _l2(q[0], q[1], ni, bg2p)
    return out[:N, :NCLS]


# speedup vs baseline: 1.0117x; 1.0117x over previous
import functools

import jax
import jax.numpy as jnp
from jax import lax
from jax.experimental import pallas as pl
from jax.experimental.pallas import tpu as pltpu
from jax.experimental.pallas import tpu_sc as plsc

N = 10000
NP = 10240
E = 640000
M = 8000
D_IN = 1000
H1 = 500
LAT = 64
DEC = 64
HID = 64
NCLS = 8

NCORES = 2
NTILES = 16
CHUNK = 128
ECHUNKS = E // CHUNK
ECHUNKS_PC = ECHUNKS // NCORES
NPT = NP // NTILES
NCHUNKS_N = NP // CHUNK


def _mm1_body(x_ref, w_ref, b_ref, u_ref, s_ref, ss_ref):
    i = pl.program_id(0)
    u = jnp.dot(x_ref[...], w_ref[...], preferred_element_type=jnp.float32,
                precision=lax.Precision.DEFAULT)
    u = u + b_ref[...]
    u_ref[...] = u

    @pl.when(i == 0)
    def _():
        s_ref[...] = jnp.zeros_like(s_ref)
        ss_ref[...] = jnp.zeros_like(ss_ref)

    s_ref[...] += jnp.sum(u, axis=0, keepdims=True)
    ss_ref[...] += jnp.sum(u * u, axis=0, keepdims=True)


def _mm1(x, w, b):
    bm = 1000
    grid = M // bm
    return pl.pallas_call(
        _mm1_body,
        grid=(grid,),
        in_specs=[
            pl.BlockSpec((bm, D_IN), lambda i: (i, 0)),
            pl.BlockSpec((D_IN, H1), lambda i: (0, 0)),
            pl.BlockSpec((1, H1), lambda i: (0, 0)),
        ],
        out_specs=[
            pl.BlockSpec((bm, H1), lambda i: (i, 0)),
            pl.BlockSpec((1, H1), lambda i: (0, 0)),
            pl.BlockSpec((1, H1), lambda i: (0, 0)),
        ],
        out_shape=[
            jax.ShapeDtypeStruct((M, H1), jnp.float32),
            jax.ShapeDtypeStruct((1, H1), jnp.float32),
            jax.ShapeDtypeStruct((1, H1), jnp.float32),
        ],
    )(x, w, b)


def _bn_scale(s, ss, g, be):
    mu = s * (1.0 / M)
    var = ss * (1.0 / M) - mu * mu
    sc = g * lax.rsqrt(var + 1e-5)
    t = be - mu * sc
    return sc, t


def _mm2_body(u_ref, w_ref, b_ref, g_ref, be_ref, s1_ref, ss1_ref,
              e_ref, s2_ref, ss2_ref):
    i = pl.program_id(0)
    sc, t = _bn_scale(s1_ref[...], ss1_ref[...], g_ref[...], be_ref[...])
    w = w_ref[...]
    bias = jnp.dot(t, w, preferred_element_type=jnp.float32) + b_ref[...]
    e = jnp.dot(u_ref[...] * sc, w, preferred_element_type=jnp.float32,
                precision=lax.Precision.DEFAULT) + bias
    e_ref[...] = e

    @pl.when(i == 0)
    def _():
        s2_ref[...] = jnp.zeros_like(s2_ref)
        ss2_ref[...] = jnp.zeros_like(ss2_ref)

    s2_ref[...] += jnp.sum(e, axis=0, keepdims=True)
    ss2_ref[...] += jnp.sum(e * e, axis=0, keepdims=True)


def _mm2(u, w, b, g, be, s1, ss1):
    bm = 1000
    grid = M // bm
    return pl.pallas_call(
        _mm2_body,
        grid=(grid,),
        in_specs=[
            pl.BlockSpec((bm, H1), lambda i: (i, 0)),
            pl.BlockSpec((H1, LAT), lambda i: (0, 0)),
            pl.BlockSpec((1, LAT), lambda i: (0, 0)),
            pl.BlockSpec((1, H1), lambda i: (0, 0)),
            pl.BlockSpec((1, H1), lambda i: (0, 0)),
            pl.BlockSpec((1, H1), lambda i: (0, 0)),
            pl.BlockSpec((1, H1), lambda i: (0, 0)),
        ],
        out_specs=[
            pl.BlockSpec((bm, LAT), lambda i: (i, 0)),
            pl.BlockSpec((1, LAT), lambda i: (0, 0)),
            pl.BlockSpec((1, LAT), lambda i: (0, 0)),
        ],
        out_shape=[
            jax.ShapeDtypeStruct((M, LAT), jnp.float32),
            jax.ShapeDtypeStruct((1, LAT), jnp.float32),
            jax.ShapeDtypeStruct((1, LAT), jnp.float32),
        ],
    )(u, w, b, g, be, s1, ss1)


_SGN = -2147483648
_LOW31 = 0x7FFFFFFF


def _order_stat(ksh, r):
    negm = jnp.where(ksh < 0, 1, 0).astype(jnp.int32)
    cneg128 = jnp.sum(negm, axis=0, keepdims=True)
    cneg = cneg128[:, :DEC] + cneg128[:, DEC:]
    in_negi = jnp.where(r < cneg, 1, 0).astype(jnp.int32)
    in_neg2i = jnp.concatenate([in_negi, in_negi], axis=1)
    grp = (in_neg2i * negm + (1 - in_neg2i) * (1 - negm)).astype(jnp.float32)
    radj = jnp.where(in_negi == 1, r, r - cneg).astype(jnp.float32)
    q = ksh & _LOW31
    p = jnp.zeros((1, DEC), jnp.int32)
    ones_row = jnp.ones((1, M // 2), jnp.float32)

    def step(i, carry):
        p, radj, cand = carry
        b = 30 - i
        bit = jnp.left_shift(jnp.int32(1), b)
        iszero = jnp.where((q & bit) == 0, 1.0, 0.0)
        m = cand * iszero
        c128 = jnp.dot(ones_row, m, preferred_element_type=jnp.float32)
        c0 = c128[:, :DEC] + c128[:, DEC:]
        take1 = radj >= c0
        take1f = jnp.where(take1, 1.0, 0.0)
        p = p | jnp.left_shift(jnp.where(take1, 1, 0).astype(jnp.int32), b)
        radj = radj - take1f * c0
        t2f = jnp.concatenate([take1f, take1f], axis=1)
        cand = cand * (iszero + t2f - 2.0 * iszero * t2f)
        return p, radj, cand

    p, radj, cand = lax.fori_loop(0, 31, step, (p, radj, grp))
    return jnp.where(in_negi == 1, p | _SGN, p)


def _mm3_body(e_ref, w_ref, b_ref, g_ref, be_ref, s2_ref, ss2_ref,
              dec_ref, med_ref):
    sc, t = _bn_scale(s2_ref[...], ss2_ref[...], g_ref[...], be_ref[...])
    w = w_ref[...]
    bias = jnp.dot(t, w, preferred_element_type=jnp.float32) + b_ref[...]
    dec = jnp.dot(e_ref[...] * sc, w, preferred_element_type=jnp.float32) + bias
    dec_ref[...] = dec

    bi = lax.bitcast_convert_type(dec, jnp.int32)
    ks = jnp.where(bi < 0, bi ^ _LOW31, bi)
    ksh = jnp.concatenate([ks[: M // 2], ks[M // 2:]], axis=1)
    k1 = _order_stat(ksh, M // 2 - 1)
    k1_2 = jnp.concatenate([k1, k1], axis=1)
    le128 = jnp.sum(jnp.where(ksh <= k1_2, 1, 0), axis=0, keepdims=True)
    cle = le128[:, :DEC] + le128[:, DEC:]
    big = jnp.where(ksh > k1_2, ksh, jnp.int32(2147483647))
    mn128 = jnp.min(big, axis=0, keepdims=True)
    mn = jnp.minimum(mn128[:, :DEC], mn128[:, DEC:])
    k2 = jnp.where(cle > M // 2, k1, mn)

    def unkey(k):
        vi = jnp.where(k < 0, k ^ _LOW31, k)
        return lax.bitcast_convert_type(vi, jnp.float32)

    med_ref[...] = (unkey(k1) + unkey(k2)) * 0.5


def _mm3(e, w, b, g, be, s2, ss2):
    return pl.pallas_call(
        _mm3_body,
        out_shape=[
            jax.ShapeDtypeStruct((M, DEC), jnp.float32),
            jax.ShapeDtypeStruct((1, DEC), jnp.float32),
        ],
    )(e, w, b, g, be, s2, ss2)


def _scale_body(do_ref, di_ref, g0_ref, g1_ref, fs_ref, no_ref, ni_ref):
    do = do_ref[...]
    di = di_ref[...]
    no = jnp.where(do > 0, lax.rsqrt(jnp.maximum(do, 1.0)), 0.0)
    ni = jnp.where(di > 0, lax.rsqrt(jnp.maximum(di, 1.0)), 0.0)
    no_ref[...] = no
    ni_ref[...] = ni
    fs_ref[...] = (g0_ref[...] + g1_ref[...]) * 0.5 * no


def _scale(do, di, g0, g1):
    return pl.pallas_call(
        _scale_body,
        out_shape=[
            jax.ShapeDtypeStruct((NP, DEC), jnp.float32),
            jax.ShapeDtypeStruct((NP, 1), jnp.float32),
            jax.ShapeDtypeStruct((NP, 1), jnp.float32),
        ],
    )(do, di, g0, g1)


def _l1_body(p0_ref, p1_ref, ni_ref, no_ref, w1_ref, b1_ref, w2_ref, y_ref):
    agg = (p0_ref[...] + p1_ref[...]) * ni_ref[...]
    h = jnp.dot(agg, w1_ref[...], preferred_element_type=jnp.float32)
    h = jax.nn.relu(h + b1_ref[...])
    y_ref[...] = jnp.dot(h * no_ref[...], w2_ref[...],
                         preferred_element_type=jnp.float32)


def _l1(p0, p1, ni, no, w1, b1, w2p):
    return pl.pallas_call(
        _l1_body,
        out_shape=jax.ShapeDtypeStruct((NP, 16), jnp.float32),
    )(p0, p1, ni, no, w1, b1, w2p)


def _l2_body(q0_ref, q1_ref, ni_ref, b_ref, o_ref):
    o_ref[...] = (q0_ref[...] + q1_ref[...]) * ni_ref[...] + b_ref[...]


def _l2(q0, q1, ni, b):
    return pl.pallas_call(
        _l2_body,
        out_shape=jax.ShapeDtypeStruct((NP, 16), jnp.float32),
    )(q0, q1, ni, b)


_MESH = dict(core_axis_name="c", subcore_axis_name="s")
_SC_PARAMS = pltpu.CompilerParams(use_tc_tiling_on_sc=False)


def _deg_gather_sc(ei3, ones128, zdeg, t0, t1, r03, r13):
    @functools.partial(
        pl.kernel,
        mesh=plsc.VectorSubcoreMesh(**_MESH),
        compiler_params=_SC_PARAMS,
        out_type=(
            jax.ShapeDtypeStruct((NCORES, NP), jnp.float32),
            jax.ShapeDtypeStruct((NP, DEC), jnp.float32),
            jax.ShapeDtypeStruct((NP, DEC), jnp.float32),
        ),
        scratch_types=[
            pltpu.VMEM((8, CHUNK), jnp.int32),
            pltpu.VMEM((1, CHUNK), jnp.int32),
            pltpu.VMEM((CHUNK,), jnp.float32),
            pltpu.VMEM((NPT,), jnp.float32),
            pltpu.VMEM((CHUNK,), jnp.int32),
            pltpu.VMEM((CHUNK,), jnp.int32),
            pltpu.VMEM((CHUNK, DEC), jnp.float32),
            pltpu.VMEM((CHUNK, DEC), jnp.float32),
            pltpu.VMEM_SHARED((NP,), jnp.float32),
            pltpu.SemaphoreType.DMA,
            pltpu.SemaphoreType.DMA,
        ],
    )
    def k(ei_hbm, ones_hbm, z_hbm, t0_hbm, t1_hbm, r0_hbm, r1_hbm,
          deg_hbm, g0_hbm, g1_hbm,
          idx_v, idx1_v, ones_v, buf_v, i0_v, i1_v, b0_v, b1_v,
          acc_sh, sem, gsem):
        c = lax.axis_index("c")
        s = lax.axis_index("s")
        pltpu.sync_copy(z_hbm.at[pl.ds(s * NPT, NPT)], buf_v)
        pltpu.sync_copy(buf_v, acc_sh.at[pl.ds(s * NPT, NPT)])
        pltpu.sync_copy(ones_hbm, ones_v)
        plsc.subcore_barrier()

        wid = s * NCORES + c
        gch = jnp.where(wid < 16, 3, 2)
        gcbase = wid * 2 + jnp.minimum(wid, 16)

        def gbody(kk, carry):
            ch = gcbase + kk
            pltpu.sync_copy(r0_hbm.at[ch], i0_v)
            pltpu.sync_copy(r1_hbm.at[ch], i1_v)
            cp0 = pltpu.async_copy(t0_hbm.at[i0_v], b0_v, gsem)
            cp1 = pltpu.async_copy(t1_hbm.at[i1_v], b1_v, gsem)
            cp0.wait()
            cp1.wait()
            pltpu.sync_copy(b0_v, g0_hbm.at[pl.ds(ch * CHUNK, CHUNK)])
            pltpu.sync_copy(b1_v, g1_hbm.at[pl.ds(ch * CHUNK, CHUNK)])
            return carry

        lax.fori_loop(0, gch, gbody, 0)

        cbase = s * 312 + jnp.minimum(s, 8)

        def body(g, carry):
            gb = cbase + g * 8
            pltpu.sync_copy(ei_hbm.at[c, pl.ds(gb, 8)], idx_v)
            cps = [
                pltpu.async_copy(ones_v, acc_sh.at[idx_v.at[j]], sem, add=True)
                for j in range(8)
            ]
            for cp in cps:
                cp.wait()
            return carry

        lax.fori_loop(0, 39, body, 0)

        @pl.when(s < 8)
        def _():
            ch = cbase + 312
            pltpu.sync_copy(ei_hbm.at[c, pl.ds(ch, 1)], idx1_v)
            pltpu.sync_copy(ones_v, acc_sh.at[idx1_v.at[0]], add=True)

        plsc.subcore_barrier()
        pltpu.sync_copy(acc_sh.at[pl.ds(s * NPT, NPT)], buf_v)
        pltpu.sync_copy(buf_v, deg_hbm.at[c, pl.ds(s * NPT, NPT)])

    return k(ei3, ones128, zdeg, t0, t1, r03, r13)


def _edge_sc(x, ei3, zeros, d):
    @functools.partial(
        pl.kernel,
        mesh=plsc.VectorSubcoreMesh(**_MESH),
        compiler_params=_SC_PARAMS,
        out_type=jax.ShapeDtypeStruct((NCORES, NP, d), jnp.float32),
        scratch_types=[
            pltpu.VMEM((6, CHUNK), jnp.int32),
            pltpu.VMEM((6, CHUNK), jnp.int32),
            pltpu.VMEM((6, CHUNK, d), jnp.float32),
            pltpu.VMEM((NPT // 4, d), jnp.float32),
            pltpu.VMEM_SHARED((NP, d), jnp.float32),
            pltpu.SemaphoreType.DMA,
            pltpu.SemaphoreType.DMA,
        ],
    )
    def k(x_hbm, ei_hbm, z_hbm, out_hbm,
          src_v, dst_v, rows_v, buf_v, acc_sh, gsem, ssem):
        c = lax.axis_index("c")
        s = lax.axis_index("s")
        zr = NPT // 4
        for t in range(4):
            pltpu.sync_copy(z_hbm.at[pl.ds(s * NPT + t * zr, zr)], buf_v)
            pltpu.sync_copy(buf_v, acc_sh.at[pl.ds(s * NPT + t * zr, zr)])
        plsc.subcore_barrier()
        cbase = c * ECHUNKS_PC + s * 156 + jnp.minimum(s, 4)

        def body(g, carry):
            gb = cbase + g * 6
            pltpu.sync_copy(ei_hbm.at[0, pl.ds(gb, 6)], src_v)
            pltpu.sync_copy(ei_hbm.at[1, pl.ds(gb, 6)], dst_v)
            gcps = [
                pltpu.async_copy(x_hbm.at[src_v.at[j]], rows_v.at[j], gsem)
                for j in range(6)
            ]
            for cp in gcps:
                cp.wait()
            scps = [
                pltpu.async_copy(rows_v.at[j], acc_sh.at[dst_v.at[j]], ssem,
                                 add=True)
                for j in range(6)
            ]
            for cp in scps:
                cp.wait()
            return carry

        lax.fori_loop(0, 26, body, 0)

        @pl.when(s < 4)
        def _():
            ch = cbase + 156
            pltpu.sync_copy(ei_hbm.at[0, pl.ds(ch, 1)], src_v.at[pl.ds(0, 1)])
            pltpu.sync_copy(ei_hbm.at[1, pl.ds(ch, 1)], dst_v.at[pl.ds(0, 1)])
            pltpu.async_copy(x_hbm.at[src_v.at[0]], rows_v.at[0], gsem).wait()
            pltpu.async_copy(rows_v.at[0], acc_sh.at[dst_v.at[0]], ssem,
                             add=True).wait()

        plsc.subcore_barrier()
        for t in range(4):
            pltpu.sync_copy(acc_sh.at[pl.ds(s * NPT + t * zr, zr)], buf_v)
            pltpu.sync_copy(buf_v, out_hbm.at[c, pl.ds(s * NPT + t * zr, zr)])

    return k(x, ei3, zeros)


def kernel(x0, x1,
           W1_0, b1_0, g1_0, be1_0, W2_0, b2_0, g2_0, be2_0, Wd_0, bd_0,
           W1_1, b1_1, g1_1, be1_1, W2_1, b2_1, g2_1, be2_1, Wd_1, bd_1,
           Wg1, bg1, Wg2, bg2, edge_index, reindex0, reindex1):
    r2 = lambda v: v.reshape(1, -1)

    tables = []
    mods = (
        (x0, (W1_0, b1_0, g1_0, be1_0, W2_0, b2_0, g2_0, be2_0, Wd_0, bd_0)),
        (x1, (W1_1, b1_1, g1_1, be1_1, W2_1, b2_1, g2_1, be2_1, Wd_1, bd_1)),
    )
    for x, (W1, b1, g1, be1, W2, b2, g2, be2, Wd, bd) in mods:
        u, s1, ss1 = _mm1(x, W1, r2(b1))
        e, s2, ss2 = _mm2(u, W2, r2(b2), r2(g1), r2(be1), s1, ss1)
        dec, med = _mm3(e, Wd, r2(bd), r2(g2), r2(be2), s2, ss2)
        t = jnp.concatenate(
            [dec, jnp.broadcast_to(med, (N - M, DEC)),
             jnp.zeros((NP - N, DEC), jnp.float32)], axis=0)
        tables.append(t)

    ei3 = edge_index.reshape(2, ECHUNKS, CHUNK)
    pad_idx = jnp.zeros((NP - N,), jnp.int32)
    r03 = jnp.concatenate([reindex0, pad_idx]).reshape(NCHUNKS_N, CHUNK)
    r13 = jnp.concatenate([reindex1, pad_idx]).reshape(NCHUNKS_N, CHUNK)

    ones128 = jnp.ones((CHUNK,), jnp.float32)
    zdeg = jnp.zeros((NP,), jnp.float32)
    z64 = jnp.zeros((NP, DEC), jnp.float32)
    z16 = jnp.zeros((NP, 16), jnp.float32)

    deg, g0, g1 = _deg_gather_sc(ei3, ones128, zdeg,
                                 tables[0], tables[1], r03, r13)
    fs, no, ni = _scale(deg[0].reshape(NP, 1), deg[1].reshape(NP, 1), g0, g1)

    p = _edge_sc(fs, ei3, z64, DEC)

    Wg2p = jnp.concatenate(
        [Wg2, jnp.zeros((HID, 16 - NCLS), jnp.float32)], axis=1)
    bg2p = jnp.concatenate(
        [bg2, jnp.zeros((16 - NCLS,), jnp.float32)]).reshape(1, 16)

    y = _l1(p[0], p[1], ni, no, Wg1, r2(bg1), Wg2p)
    q = _edge_sc(y, ei3, z16, 16)
    out = _l2(q[0], q[1], ni, bg2p)
    return out[:N, :NCLS]
